# baseline (speedup 1.0000x reference)
import jax
import jax.numpy as jnp
from jax import lax
from jax.experimental import pallas as pl
from jax.experimental.pallas import tpu as pltpu
from jax.experimental.pallas import tpu_sc as plsc

CHUNKS = 8
RES_DIM = 2048
BIAS = 1.6
C1, C3, C5 = 1.0, -1.0 / 3.0, 2.0 / 15.0

NC, NS, L = 2, 16, 16
NW = NC * NS
TOTAL_ROWS = CHUNKS * RES_DIM
RPW = TOTAL_ROWS // NW
GROUP = 16
NGRP = RPW // GROUP
JBLK = RES_DIM // L


def _poly(pre):
    p2 = pre * pre
    return pre * (C1 + p2 * (C3 + p2 * C5))


def _sc_body(pv_hbm, s_hbm, wr_hbm, out_hbm,
             wrbuf, sbuf, pvbuf, obuf, tbuf, sem0, sem1):
    wid = lax.axis_index("s") * NC + lax.axis_index("c")
    row0 = wid * RPW
    chunk = row0 // RES_DIM

    pltpu.sync_copy(s_hbm.at[pl.ds(chunk * RES_DIM, RES_DIM)], sbuf)
    pltpu.sync_copy(pv_hbm.at[pl.ds(row0, RPW)], pvbuf)

    sems = (sem0, sem1)
    iota16 = lax.iota(jnp.int32, L)
    zeros = jnp.zeros((L,), jnp.float32)
    tbuf[pl.ds(0, L)] = zeros
    tbuf[pl.ds(2 * L, L)] = zeros

    def shift_dn(v, k):
        tbuf[pl.ds(L, L)] = v
        return tbuf[pl.ds(L + k, L)]

    def shift_up(v, k):
        tbuf[pl.ds(L, L)] = v
        return tbuf[pl.ds(L - k, L)]

    def bitrev4(i):
        return ((i & 1) << 3) | ((i & 2) << 1) | ((i & 4) >> 1) | ((i & 8) >> 3)

    def start(g):
        b = g % 2
        return pltpu.async_copy(
            wr_hbm.at[pl.ds(row0 + g * GROUP, GROUP)], wrbuf.at[b], sems[b])

    handles = {0: start(0)}

    for g in range(NGRP):
        b = g % 2
        handles.pop(g).wait()
        if g + 1 < NGRP:
            handles[g + 1] = start(g + 1)

        def jb_body(jb, accs):
            base = jb * L
            sv = sbuf[pl.ds(base, L)]
            return tuple(accs[r] + wrbuf[b, r, pl.ds(base, L)] * sv
                         for r in range(GROUP))

        accs = lax.fori_loop(
            0, JBLK, jb_body,
            tuple(jnp.zeros((L,), jnp.float32) for _ in range(GROUP)))

        vecs = [accs[bitrev4(i)] for i in range(GROUP)]
        k = L // 2
        while len(vecs) > 1:
            keep = (iota16 & k) == 0
            nxt = []
            for i in range(0, len(vecs), 2):
                a = vecs[i] + shift_dn(vecs[i], k)
                b = vecs[i + 1] + shift_dn(vecs[i + 1], k)
                nxt.append(jnp.where(keep, a, shift_up(b, k)))
            vecs = nxt
            k //= 2
        dots = vecs[0]

        pre = dots + pvbuf[pl.ds(g * GROUP, L)] + BIAS
        obuf[pl.ds(g * GROUP, L)] = _poly(pre)

    pltpu.sync_copy(obuf, out_hbm.at[pl.ds(row0, RPW)])


def kernel(proj_vars, res_state, wr):
    mesh = plsc.VectorSubcoreMesh(core_axis_name="c", subcore_axis_name="s")
    sc_call = pl.kernel(
        _sc_body,
        mesh=mesh,
        out_type=jax.ShapeDtypeStruct((TOTAL_ROWS,), jnp.float32),
        scratch_types=[
            pltpu.VMEM((2, GROUP, RES_DIM), jnp.float32),
            pltpu.VMEM((RES_DIM,), jnp.float32),
            pltpu.VMEM((RPW,), jnp.float32),
            pltpu.VMEM((RPW,), jnp.float32),
            pltpu.VMEM((GROUP * L,), jnp.float32),
            pltpu.SemaphoreType.DMA,
            pltpu.SemaphoreType.DMA,
        ],
    )
    out = sc_call(proj_vars.reshape(-1), res_state.reshape(-1),
                  wr.reshape(TOTAL_ROWS, RES_DIM))
    return out.reshape(CHUNKS, RES_DIM)
